# baseline (speedup 1.0000x reference)
import functools

import jax
import jax.numpy as jnp
from jax import lax
from jax.experimental import pallas as pl
from jax.experimental.pallas import tpu as pltpu
from jax.experimental.pallas import tpu_sc as plsc

T = 2048
H = 1024
HW = H // 2
I = 512
E = 8
TOPK = 2
MB = 256
S_PAD = T * TOPK + E * MB
NB = S_PAD // MB

NC, NS, L = 2, 16, 16
NW = NC * NS
TPW = T // NW

_MESH = plsc.VectorSubcoreMesh(core_axis_name="c", subcore_axis_name="s")


def _routing(x, gate_w, corr_bias):
    router_logits = x.astype(jnp.float32) @ gate_w
    routing_weights = jax.nn.softmax(router_logits, axis=1)
    scores = routing_weights + corr_bias.squeeze()
    _, selected_experts = jax.lax.top_k(scores, TOPK)
    onehot = jax.nn.one_hot(selected_experts, E, dtype=jnp.float32)
    w = jnp.sum(onehot * routing_weights[:, None, :], axis=2)
    w = w / jnp.clip(jnp.sum(w, axis=-1, keepdims=True), 1e-12)
    return router_logits, onehot, w


def _positions(onehot):
    ohi = onehot.astype(jnp.int32)
    ce = jnp.sum(ohi, axis=1)
    excl = jnp.cumsum(ce, axis=0) - ce
    counts = jnp.sum(ce, axis=0)
    starts = []
    cur = jnp.int32(0)
    for e in range(E):
        starts.append(cur)
        cur = ((cur + counts[e] + MB - 1) // MB) * MB
    starts = jnp.stack(starts)
    position = jnp.sum(ohi * (starts[None, None, :] + excl[:, None, :]), axis=2)
    blk = jnp.arange(NB, dtype=jnp.int32)[:, None] * MB
    block_to_expert = jnp.sum((starts[None, :] <= blk).astype(jnp.int32), axis=1) - 1
    return position, block_to_expert


HALF = TPW // 2


@functools.partial(
    pl.kernel,
    out_type=jax.ShapeDtypeStruct((S_PAD, H), jnp.float32),
    mesh=_MESH,
    scratch_types=[
        pltpu.VMEM((2, HALF), jnp.int32),
        pltpu.VMEM((2, HALF), jnp.int32),
        pltpu.VMEM((HALF, H), jnp.float32),
        pltpu.VMEM((HALF, H), jnp.float32),
        pltpu.SemaphoreType.DMA,
        pltpu.SemaphoreType.DMA,
    ],
)
def _dispatch(x_hbm, pos0_hbm, pos1_hbm, xs_hbm, idx0_v, idx1_v, ra_v, rb_v,
              lsem, ssem):
    wid = lax.axis_index("s") * NC + lax.axis_index("c")
    base = wid * TPW
    pltpu.sync_copy(pos0_hbm.at[pl.ds(2 * wid, 2)], idx0_v)
    pltpu.sync_copy(pos1_hbm.at[pl.ds(2 * wid, 2)], idx1_v)
    la = pltpu.async_copy(x_hbm.at[pl.ds(base, HALF)], ra_v, lsem)
    lb = pltpu.async_copy(x_hbm.at[pl.ds(base + HALF, HALF)], rb_v, lsem)
    la.wait()
    s0 = pltpu.async_copy(ra_v, xs_hbm.at[idx0_v.at[0]], ssem)
    s1 = pltpu.async_copy(ra_v, xs_hbm.at[idx1_v.at[0]], ssem)
    lb.wait()
    s2 = pltpu.async_copy(rb_v, xs_hbm.at[idx0_v.at[1]], ssem)
    s3 = pltpu.async_copy(rb_v, xs_hbm.at[idx1_v.at[1]], ssem)
    s0.wait()
    s1.wait()
    s2.wait()
    s3.wait()


@functools.partial(
    pl.kernel,
    out_type=(
        jax.ShapeDtypeStruct((T, H), jnp.float32),
        jax.ShapeDtypeStruct((T, H), jnp.float32),
    ),
    mesh=_MESH,
    scratch_types=[
        pltpu.VMEM((2, HALF), jnp.int32),
        pltpu.VMEM((2, HALF), jnp.int32),
        pltpu.VMEM((HALF, H), jnp.float32),
        pltpu.VMEM((HALF, H), jnp.float32),
        pltpu.SemaphoreType.DMA,
    ],
)
def _zgather(ys_hbm, pos0_hbm, pos1_hbm, z0_hbm, z1_hbm,
             idx0_v, idx1_v, ra_v, rb_v, sem):
    wid = lax.axis_index("s") * NC + lax.axis_index("c")
    base = wid * TPW
    pltpu.sync_copy(pos0_hbm.at[pl.ds(2 * wid, 2)], idx0_v)
    pltpu.sync_copy(pos1_hbm.at[pl.ds(2 * wid, 2)], idx1_v)
    ga = pltpu.async_copy(ys_hbm.at[idx0_v.at[0]], ra_v, sem)
    gb = pltpu.async_copy(ys_hbm.at[idx0_v.at[1]], rb_v, sem)
    ga.wait()
    pltpu.sync_copy(ra_v, z0_hbm.at[pl.ds(base, HALF)])
    gc = pltpu.async_copy(ys_hbm.at[idx1_v.at[0]], ra_v, sem)
    gb.wait()
    pltpu.sync_copy(rb_v, z0_hbm.at[pl.ds(base + HALF, HALF)])
    gd = pltpu.async_copy(ys_hbm.at[idx1_v.at[1]], rb_v, sem)
    gc.wait()
    pltpu.sync_copy(ra_v, z1_hbm.at[pl.ds(base, HALF)])
    gd.wait()
    pltpu.sync_copy(rb_v, z1_hbm.at[pl.ds(base + HALF, HALF)])


def _group_mlp_body(e_ref, xs_ref, wg_ref, wu_ref, wd_ref, ys_ref):
    xb = xs_ref[...].astype(jnp.bfloat16)
    g = jnp.dot(xb, wg_ref[0].astype(jnp.bfloat16), preferred_element_type=jnp.float32)
    u = jnp.dot(xb, wu_ref[0].astype(jnp.bfloat16), preferred_element_type=jnp.float32)
    h = (g * jax.nn.sigmoid(g) * u).astype(jnp.bfloat16)
    y = jnp.dot(h, wd_ref[0].astype(jnp.bfloat16), preferred_element_type=jnp.float32)
    ys_ref[...] = y


def _shared_combine_body(x_ref, sg_ref, su_ref, sd_ref, z0_ref, z1_ref, w0_ref, w1_ref, out_ref):
    xb = x_ref[...].astype(jnp.bfloat16)
    g = jnp.dot(xb, sg_ref[...].astype(jnp.bfloat16), preferred_element_type=jnp.float32)
    u = jnp.dot(xb, su_ref[...].astype(jnp.bfloat16), preferred_element_type=jnp.float32)
    h = (g * jax.nn.sigmoid(g) * u).astype(jnp.bfloat16)
    y = jnp.dot(h, sd_ref[...].astype(jnp.bfloat16), preferred_element_type=jnp.float32)
    out_ref[...] = y + w0_ref[:, 0:1] * z0_ref[...] + w1_ref[:, 0:1] * z1_ref[...]


def kernel(hidden_states, gate_w, corr_bias, Wg, Wu, Wd, Sg, Su, Sd):
    b, s, h = hidden_states.shape
    x = hidden_states.reshape(T, H)

    router_logits, onehot, w = _routing(x, gate_w, corr_bias)
    position, block_to_expert = _positions(onehot)
    pos0 = position[:, 0].astype(jnp.int32).reshape(T // HALF, HALF)
    pos1 = position[:, 1].astype(jnp.int32).reshape(T // HALF, HALF)
    w0_rep = jnp.broadcast_to(w[:, 0:1], (T, 128)).astype(jnp.float32)
    w1_rep = jnp.broadcast_to(w[:, 1:2], (T, 128)).astype(jnp.float32)

    xs = _dispatch(x, pos0, pos1)

    grid_spec = pltpu.PrefetchScalarGridSpec(
        num_scalar_prefetch=1,
        grid=(NB,),
        in_specs=[
            pl.BlockSpec((MB, H), lambda m, e_ref: (m, 0)),
            pl.BlockSpec((1, H, I), lambda m, e_ref: (e_ref[m], 0, 0)),
            pl.BlockSpec((1, H, I), lambda m, e_ref: (e_ref[m], 0, 0)),
            pl.BlockSpec((1, I, H), lambda m, e_ref: (e_ref[m], 0, 0)),
        ],
        out_specs=pl.BlockSpec((MB, H), lambda m, e_ref: (m, 0)),
    )
    ys = pl.pallas_call(
        _group_mlp_body,
        grid_spec=grid_spec,
        out_shape=jax.ShapeDtypeStruct((S_PAD, H), jnp.float32),
    )(block_to_expert, xs, Wg, Wu, Wd)

    z0, z1 = _zgather(ys, pos0, pos1)

    BT = 1024
    out = pl.pallas_call(
        _shared_combine_body,
        grid=(T // BT,),
        in_specs=[
            pl.BlockSpec((BT, H), lambda i: (i, 0)),
            pl.BlockSpec((H, H), lambda i: (0, 0)),
            pl.BlockSpec((H, H), lambda i: (0, 0)),
            pl.BlockSpec((H, H), lambda i: (0, 0)),
            pl.BlockSpec((BT, H), lambda i: (i, 0)),
            pl.BlockSpec((BT, H), lambda i: (i, 0)),
            pl.BlockSpec((BT, 128), lambda i: (i, 0)),
            pl.BlockSpec((BT, 128), lambda i: (i, 0)),
        ],
        out_specs=pl.BlockSpec((BT, H), lambda i: (i, 0)),
        out_shape=jax.ShapeDtypeStruct((T, H), jnp.float32),
    )(x, Sg, Su, Sd, z0, z1, w0_rep, w1_rep)

    return out.reshape(b, s, h), router_logits


# speedup vs baseline: 1.0088x; 1.0088x over previous
import functools

import jax
import jax.numpy as jnp
from jax import lax
from jax.experimental import pallas as pl
from jax.experimental.pallas import tpu as pltpu
from jax.experimental.pallas import tpu_sc as plsc

T = 2048
H = 1024
HW = H // 2
I = 512
E = 8
TOPK = 2
MB = 256
S_PAD = T * TOPK + E * MB
NB = S_PAD // MB

NC, NS, L = 2, 16, 16
NW = NC * NS
TPW = T // NW

_MESH = plsc.VectorSubcoreMesh(core_axis_name="c", subcore_axis_name="s")


def _routing(x, gate_w, corr_bias):
    router_logits = x.astype(jnp.float32) @ gate_w
    routing_weights = jax.nn.softmax(router_logits, axis=1)
    scores = routing_weights + corr_bias.squeeze()
    _, selected_experts = jax.lax.top_k(scores, TOPK)
    onehot = jax.nn.one_hot(selected_experts, E, dtype=jnp.float32)
    w = jnp.sum(onehot * routing_weights[:, None, :], axis=2)
    w = w / jnp.clip(jnp.sum(w, axis=-1, keepdims=True), 1e-12)
    return router_logits, onehot, w


def _positions(onehot):
    ohi = onehot.astype(jnp.int32)
    ce = jnp.sum(ohi, axis=1)
    excl = jnp.cumsum(ce, axis=0) - ce
    counts = jnp.sum(ce, axis=0)
    starts = []
    cur = jnp.int32(0)
    for e in range(E):
        starts.append(cur)
        cur = ((cur + counts[e] + MB - 1) // MB) * MB
    starts = jnp.stack(starts)
    position = jnp.sum(ohi * (starts[None, None, :] + excl[:, None, :]), axis=2)
    blk = jnp.arange(NB, dtype=jnp.int32)[:, None] * MB
    block_to_expert = jnp.sum((starts[None, :] <= blk).astype(jnp.int32), axis=1) - 1
    group_end = (starts + counts)[block_to_expert]
    block_valid = (blk[:, 0] < group_end).astype(jnp.int32)
    return position, block_to_expert, block_valid


HALF = TPW // 2


@functools.partial(
    pl.kernel,
    out_type=jax.ShapeDtypeStruct((S_PAD, H), jnp.float32),
    mesh=_MESH,
    scratch_types=[
        pltpu.VMEM((2, HALF), jnp.int32),
        pltpu.VMEM((2, HALF), jnp.int32),
        pltpu.VMEM((HALF, H), jnp.float32),
        pltpu.VMEM((HALF, H), jnp.float32),
        pltpu.SemaphoreType.DMA,
        pltpu.SemaphoreType.DMA,
    ],
)
def _dispatch(x_hbm, pos0_hbm, pos1_hbm, xs_hbm, idx0_v, idx1_v, ra_v, rb_v,
              lsem, ssem):
    wid = lax.axis_index("s") * NC + lax.axis_index("c")
    base = wid * TPW
    pltpu.sync_copy(pos0_hbm.at[pl.ds(2 * wid, 2)], idx0_v)
    pltpu.sync_copy(pos1_hbm.at[pl.ds(2 * wid, 2)], idx1_v)
    la = pltpu.async_copy(x_hbm.at[pl.ds(base, HALF)], ra_v, lsem)
    lb = pltpu.async_copy(x_hbm.at[pl.ds(base + HALF, HALF)], rb_v, lsem)
    la.wait()
    s0 = pltpu.async_copy(ra_v, xs_hbm.at[idx0_v.at[0]], ssem)
    s1 = pltpu.async_copy(ra_v, xs_hbm.at[idx1_v.at[0]], ssem)
    lb.wait()
    s2 = pltpu.async_copy(rb_v, xs_hbm.at[idx0_v.at[1]], ssem)
    s3 = pltpu.async_copy(rb_v, xs_hbm.at[idx1_v.at[1]], ssem)
    s0.wait()
    s1.wait()
    s2.wait()
    s3.wait()


@functools.partial(
    pl.kernel,
    out_type=(
        jax.ShapeDtypeStruct((T, H), jnp.float32),
        jax.ShapeDtypeStruct((T, H), jnp.float32),
    ),
    mesh=_MESH,
    scratch_types=[
        pltpu.VMEM((2, HALF), jnp.int32),
        pltpu.VMEM((2, HALF), jnp.int32),
        pltpu.VMEM((HALF, H), jnp.float32),
        pltpu.VMEM((HALF, H), jnp.float32),
        pltpu.SemaphoreType.DMA,
    ],
)
def _zgather(ys_hbm, pos0_hbm, pos1_hbm, z0_hbm, z1_hbm,
             idx0_v, idx1_v, ra_v, rb_v, sem):
    wid = lax.axis_index("s") * NC + lax.axis_index("c")
    base = wid * TPW
    pltpu.sync_copy(pos0_hbm.at[pl.ds(2 * wid, 2)], idx0_v)
    pltpu.sync_copy(pos1_hbm.at[pl.ds(2 * wid, 2)], idx1_v)
    ga = pltpu.async_copy(ys_hbm.at[idx0_v.at[0]], ra_v, sem)
    gb = pltpu.async_copy(ys_hbm.at[idx0_v.at[1]], rb_v, sem)
    ga.wait()
    pltpu.sync_copy(ra_v, z0_hbm.at[pl.ds(base, HALF)])
    gc = pltpu.async_copy(ys_hbm.at[idx1_v.at[0]], ra_v, sem)
    gb.wait()
    pltpu.sync_copy(rb_v, z0_hbm.at[pl.ds(base + HALF, HALF)])
    gd = pltpu.async_copy(ys_hbm.at[idx1_v.at[1]], rb_v, sem)
    gc.wait()
    pltpu.sync_copy(ra_v, z1_hbm.at[pl.ds(base, HALF)])
    gd.wait()
    pltpu.sync_copy(rb_v, z1_hbm.at[pl.ds(base + HALF, HALF)])


def _group_mlp_body(e_ref, v_ref, xs_ref, wg_ref, wu_ref, wd_ref, ys_ref):
    @pl.when(v_ref[pl.program_id(0)] != 0)
    def _():
        xb = xs_ref[...].astype(jnp.bfloat16)
        g = jnp.dot(xb, wg_ref[0].astype(jnp.bfloat16), preferred_element_type=jnp.float32)
        u = jnp.dot(xb, wu_ref[0].astype(jnp.bfloat16), preferred_element_type=jnp.float32)
        h = (g * jax.nn.sigmoid(g) * u).astype(jnp.bfloat16)
        y = jnp.dot(h, wd_ref[0].astype(jnp.bfloat16), preferred_element_type=jnp.float32)
        ys_ref[...] = y


def _shared_combine_body(x_ref, sg_ref, su_ref, sd_ref, z0_ref, z1_ref, w0_ref, w1_ref, out_ref):
    xb = x_ref[...].astype(jnp.bfloat16)
    g = jnp.dot(xb, sg_ref[...].astype(jnp.bfloat16), preferred_element_type=jnp.float32)
    u = jnp.dot(xb, su_ref[...].astype(jnp.bfloat16), preferred_element_type=jnp.float32)
    h = (g * jax.nn.sigmoid(g) * u).astype(jnp.bfloat16)
    y = jnp.dot(h, sd_ref[...].astype(jnp.bfloat16), preferred_element_type=jnp.float32)
    out_ref[...] = y + w0_ref[:, 0:1] * z0_ref[...] + w1_ref[:, 0:1] * z1_ref[...]


def kernel(hidden_states, gate_w, corr_bias, Wg, Wu, Wd, Sg, Su, Sd):
    b, s, h = hidden_states.shape
    x = hidden_states.reshape(T, H)

    router_logits, onehot, w = _routing(x, gate_w, corr_bias)
    position, block_to_expert, block_valid = _positions(onehot)
    pos0 = position[:, 0].astype(jnp.int32).reshape(T // HALF, HALF)
    pos1 = position[:, 1].astype(jnp.int32).reshape(T // HALF, HALF)
    w0_rep = jnp.broadcast_to(w[:, 0:1], (T, 128)).astype(jnp.float32)
    w1_rep = jnp.broadcast_to(w[:, 1:2], (T, 128)).astype(jnp.float32)

    xs = _dispatch(x, pos0, pos1)

    grid_spec = pltpu.PrefetchScalarGridSpec(
        num_scalar_prefetch=2,
        grid=(NB,),
        in_specs=[
            pl.BlockSpec((MB, H), lambda m, e_ref, v_ref: (m, 0)),
            pl.BlockSpec((1, H, I), lambda m, e_ref, v_ref: (e_ref[m], 0, 0)),
            pl.BlockSpec((1, H, I), lambda m, e_ref, v_ref: (e_ref[m], 0, 0)),
            pl.BlockSpec((1, I, H), lambda m, e_ref, v_ref: (e_ref[m], 0, 0)),
        ],
        out_specs=pl.BlockSpec((MB, H), lambda m, e_ref, v_ref: (m, 0)),
    )
    ys = pl.pallas_call(
        _group_mlp_body,
        grid_spec=grid_spec,
        out_shape=jax.ShapeDtypeStruct((S_PAD, H), jnp.float32),
    )(block_to_expert, block_valid, xs, Wg, Wu, Wd)

    z0, z1 = _zgather(ys, pos0, pos1)

    BT = 1024
    out = pl.pallas_call(
        _shared_combine_body,
        grid=(T // BT,),
        in_specs=[
            pl.BlockSpec((BT, H), lambda i: (i, 0)),
            pl.BlockSpec((H, H), lambda i: (0, 0)),
            pl.BlockSpec((H, H), lambda i: (0, 0)),
            pl.BlockSpec((H, H), lambda i: (0, 0)),
            pl.BlockSpec((BT, H), lambda i: (i, 0)),
            pl.BlockSpec((BT, H), lambda i: (i, 0)),
            pl.BlockSpec((BT, 128), lambda i: (i, 0)),
            pl.BlockSpec((BT, 128), lambda i: (i, 0)),
        ],
        out_specs=pl.BlockSpec((BT, H), lambda i: (i, 0)),
        out_shape=jax.ShapeDtypeStruct((T, H), jnp.float32),
    )(x, Sg, Su, Sd, z0, z1, w0_rep, w1_rep)

    return out.reshape(b, s, h), router_logits
